# baseline (speedup 1.0000x reference)
import dataclasses
import functools

import jax
import jax.numpy as jnp
from jax import lax
from jax.experimental import pallas as pl
from jax.experimental.pallas import tpu as pltpu
from jax.experimental.pallas import tpu_sc as plsc

_N_USERS = 5000
_N = 10000
_D = 128
_DH = _D // 2
_E = 320000
_NC = 2
_NS = 16
_EPT = _E // _NS
_K = 128
_EPW = 20480
_NCH = _EPW // _K
_NP = 10240
_RPS = _NP // _NS
_ZR = 128
_NBUF = 5

_mesh = plsc.VectorSubcoreMesh(core_axis_name="c", subcore_axis_name="s")

_sc_params = dataclasses.replace(
    pltpu.CompilerParams(),
    needs_layout_passes=False,
    use_tc_tiling_on_sc=False,
)


@functools.partial(
    pl.kernel,
    out_type=jax.ShapeDtypeStruct((_NC, _NP, _DH), jnp.bfloat16),
    mesh=_mesh,
    scratch_types=[
        pltpu.VMEM((_NCH, _K), jnp.int32),
        pltpu.VMEM((_NCH, _K), jnp.int32),
        pltpu.VMEM((_EPW,), jnp.float32),
        [pltpu.VMEM((_K, _DH), jnp.bfloat16)] * _NBUF,
        pltpu.VMEM_SHARED((_NP, _DH), jnp.bfloat16),
        [pltpu.SemaphoreType.DMA] * _NBUF,
        [pltpu.SemaphoreType.DMA] * _NBUF,
    ],
    compiler_params=_sc_params,
)
def _sc_layer(emb_hbm, col_hbm, row_hbm, val_hbm, out_hbm,
              col_v, row_v, val_v, gbufs, acc, gsems, ssems):
    c = lax.axis_index("c")
    s = lax.axis_index("s")
    emb = emb_hbm.at[c]

    pltpu.sync_copy(col_hbm.at[s], col_v)
    pltpu.sync_copy(row_hbm.at[s], row_v)
    pltpu.sync_copy(val_hbm.at[s], val_v)

    for b in range(_NBUF - 2):
        pltpu.async_copy(emb.at[col_v.at[b]], gbufs[b], gsems[b])

    zbuf = gbufs[_NBUF - 1]
    zero = jnp.zeros((32,), jnp.bfloat16)

    @pl.loop(0, _K)
    def _zrow(i):
        @pl.loop(0, _DH // 32)
        def _zseg(j):
            zbuf[i, pl.ds(j * 32, 32)] = zero

    r0 = s * _RPS

    @pl.loop(0, _RPS // _K)
    def _zcopy(t):
        pltpu.sync_copy(zbuf, acc.at[pl.ds(r0 + t * _K, _K)])

    plsc.subcore_barrier()

    @pl.loop(0, _NCH, step=_NBUF)
    def _grp(j):
        for bi in range(_NBUF):
            jc = j + bi
            gbuf = gbufs[bi]
            pltpu.make_async_copy(emb.at[col_v.at[jc]], gbuf, gsems[bi]).wait()

            @pl.loop(0, _K, step=8)
            def _edge(k0, jc=jc, gbuf=gbuf):
                vbase = jnp.full((16,), jc * _K + k0, jnp.int32)
                valbs = []
                for dk in range(8):
                    val = plsc.load_gather(val_v, [vbase + jnp.int32(dk)])
                    valbs.append(
                        plsc.pack(val, val, format=plsc.PackFormat.INTERLEAVED))
                rows = [[gbuf[k0 + dk, pl.ds(seg * 32, 32)]
                         for seg in range(_DH // 32)] for dk in range(8)]
                for dk in range(8):
                    for seg in range(_DH // 32):
                        gbuf[k0 + dk, pl.ds(seg * 32, 32)] = (
                            rows[dk][seg] * valbs[dk])

            pltpu.async_copy(gbuf, acc.at[row_v.at[jc]], ssems[bi], add=True)

            bf = (bi + _NBUF - 2) % _NBUF

            @pl.when(jc >= 2)
            def _wait_prev_scatter(jc=jc, bf=bf):
                pltpu.make_async_copy(
                    gbufs[bf], acc.at[row_v.at[jc - 2]], ssems[bf]).wait()

            @pl.when(jc + (_NBUF - 2) < _NCH)
            def _next_gather(jc=jc, bf=bf):
                pltpu.async_copy(
                    emb.at[col_v.at[jc + _NBUF - 2]], gbufs[bf], gsems[bf])

    for jc in (_NCH - 2, _NCH - 1):
        bi = jc % _NBUF
        pltpu.make_async_copy(gbufs[bi], acc.at[row_v.at[jc]], ssems[bi]).wait()
    plsc.subcore_barrier()
    pltpu.sync_copy(acc.at[pl.ds(r0, _RPS)], out_hbm.at[c].at[pl.ds(r0, _RPS)])


def _final_body(e0_ref, e1_ref, e2_ref, e3_ref, o_ref):
    sums = [(e0_ref[h]
             + e1_ref[h].astype(jnp.float32)
             + e2_ref[h].astype(jnp.float32)
             + e3_ref[h].astype(jnp.float32)) * 0.25
            for h in range(_NC)]
    o_ref[...] = jnp.concatenate(sums, axis=-1)


_final = pl.pallas_call(
    _final_body,
    out_shape=jax.ShapeDtypeStruct((_NP, _D), jnp.float32),
)


def kernel(all_users, all_items, graph_indices, graph_values):
    pad = jnp.zeros((_NP - _N, _D), jnp.float32)
    e0 = jnp.concatenate([all_users, all_items, pad], axis=0)
    e0s = jnp.stack([e0[:, :_DH], e0[:, _DH:]])
    npad = _EPW - _EPT
    col = graph_indices[1].astype(jnp.int32).reshape(_NS, _EPT)
    col = jnp.pad(col, ((0, 0), (0, npad))).reshape(_NS, _NCH, _K)
    row = graph_indices[0].astype(jnp.int32).reshape(_NS, _EPT)
    row = jnp.pad(row, ((0, 0), (0, npad)),
                  constant_values=_N).reshape(_NS, _NCH, _K)
    val = jnp.pad(graph_values.reshape(_NS, _EPT), ((0, 0), (0, npad)))

    e1s = _sc_layer(e0s.astype(jnp.bfloat16), col, row, val)
    e2s = _sc_layer(e1s, col, row, val)
    e3s = _sc_layer(e2s, col, row, val)
    light_out = _final(e0s, e1s, e2s, e3s)

    return light_out[:_N_USERS], light_out[_N_USERS:_N]


# speedup vs baseline: 1.8564x; 1.8564x over previous
import dataclasses
import functools

import jax
import jax.numpy as jnp
from jax import lax
from jax.experimental import pallas as pl
from jax.experimental.pallas import tpu as pltpu
from jax.experimental.pallas import tpu_sc as plsc

_N_USERS = 5000
_N = 10000
_D = 128
_DH = _D // 2
_E = 320000
_NC = 2
_NS = 16
_EPW = _E // _NS
_K = 80
_NCH = _EPW // _K
_NP = 10240
_RPS = _NP // _NS
_ZR = 128
_NBUF = 10

_mesh = plsc.VectorSubcoreMesh(core_axis_name="c", subcore_axis_name="s")

_sc_params = dataclasses.replace(
    pltpu.CompilerParams(),
    needs_layout_passes=False,
    use_tc_tiling_on_sc=False,
)


@functools.partial(
    pl.kernel,
    out_type=jax.ShapeDtypeStruct((_NC, _NP, _DH), jnp.bfloat16),
    mesh=_mesh,
    scratch_types=[
        pltpu.VMEM((_NCH, _K), jnp.int32),
        pltpu.VMEM((_NCH, _K), jnp.int32),
        pltpu.VMEM((_EPW,), jnp.float32),
        [pltpu.VMEM((_K, _DH), jnp.bfloat16)] * _NBUF,
        pltpu.VMEM_SHARED((_NP, _DH), jnp.bfloat16),
        [pltpu.SemaphoreType.DMA] * _NBUF,
        [pltpu.SemaphoreType.DMA] * _NBUF,
    ],
    compiler_params=_sc_params,
)
def _sc_layer(emb_hbm, col_hbm, row_hbm, val_hbm, out_hbm,
              col_v, row_v, val_v, gbufs, acc, gsems, ssems):
    c = lax.axis_index("c")
    s = lax.axis_index("s")
    emb = emb_hbm.at[c]

    pltpu.sync_copy(col_hbm.at[s], col_v)
    pltpu.sync_copy(row_hbm.at[s], row_v)
    pltpu.sync_copy(val_hbm.at[s], val_v)

    for b in range(_NBUF - 2):
        pltpu.async_copy(emb.at[col_v.at[b]], gbufs[b], gsems[b])

    zbuf = gbufs[_NBUF - 1]
    zero = jnp.zeros((32,), jnp.bfloat16)

    @pl.loop(0, _K)
    def _zrow(i):
        @pl.loop(0, _DH // 32)
        def _zseg(j):
            zbuf[i, pl.ds(j * 32, 32)] = zero

    r0 = s * _RPS

    @pl.loop(0, _RPS // _K)
    def _zcopy(t):
        pltpu.sync_copy(zbuf, acc.at[pl.ds(r0 + t * _K, _K)])

    plsc.subcore_barrier()

    @pl.loop(0, _NCH, step=_NBUF)
    def _grp(j):
        for bi in range(_NBUF):
            jc = j + bi
            gbuf = gbufs[bi]
            pltpu.make_async_copy(emb.at[col_v.at[jc]], gbuf, gsems[bi]).wait()

            @pl.loop(0, _K, step=8)
            def _edge(k0, jc=jc, gbuf=gbuf):
                vbase = jnp.full((16,), jc * _K + k0, jnp.int32)
                valbs = []
                for dk in range(8):
                    val = plsc.load_gather(val_v, [vbase + jnp.int32(dk)])
                    valbs.append(
                        plsc.pack(val, val, format=plsc.PackFormat.INTERLEAVED))
                rows = [[gbuf[k0 + dk, pl.ds(seg * 32, 32)]
                         for seg in range(_DH // 32)] for dk in range(8)]
                for dk in range(8):
                    for seg in range(_DH // 32):
                        gbuf[k0 + dk, pl.ds(seg * 32, 32)] = (
                            rows[dk][seg] * valbs[dk])

            pltpu.async_copy(gbuf, acc.at[row_v.at[jc]], ssems[bi], add=True)

            bf = (bi + _NBUF - 2) % _NBUF

            @pl.when(jc >= 2)
            def _wait_prev_scatter(jc=jc, bf=bf):
                pltpu.make_async_copy(
                    gbufs[bf], acc.at[row_v.at[jc - 2]], ssems[bf]).wait()

            @pl.when(jc + (_NBUF - 2) < _NCH)
            def _next_gather(jc=jc, bf=bf):
                pltpu.async_copy(
                    emb.at[col_v.at[jc + _NBUF - 2]], gbufs[bf], gsems[bf])

    for jc in (_NCH - 2, _NCH - 1):
        bi = jc % _NBUF
        pltpu.make_async_copy(gbufs[bi], acc.at[row_v.at[jc]], ssems[bi]).wait()
    plsc.subcore_barrier()
    pltpu.sync_copy(acc.at[pl.ds(r0, _RPS)], out_hbm.at[c].at[pl.ds(r0, _RPS)])


def _final_body(e0_ref, e1_ref, e2_ref, e3_ref, o_ref):
    sums = [(e0_ref[h]
             + e1_ref[h].astype(jnp.float32)
             + e2_ref[h].astype(jnp.float32)
             + e3_ref[h].astype(jnp.float32)) * 0.25
            for h in range(_NC)]
    o_ref[...] = jnp.concatenate(sums, axis=-1)


_final = pl.pallas_call(
    _final_body,
    out_shape=jax.ShapeDtypeStruct((_NP, _D), jnp.float32),
)


def kernel(all_users, all_items, graph_indices, graph_values):
    pad = jnp.zeros((_NP - _N, _D), jnp.float32)
    e0 = jnp.concatenate([all_users, all_items, pad], axis=0)
    e0s = jnp.stack([e0[:, :_DH], e0[:, _DH:]])
    col = graph_indices[1].astype(jnp.int32).reshape(_NS, _NCH, _K)
    row = graph_indices[0].astype(jnp.int32).reshape(_NS, _NCH, _K)
    val = graph_values.reshape(_NS, _EPW)

    e1s = _sc_layer(e0s.astype(jnp.bfloat16), col, row, val)
    e2s = _sc_layer(e1s, col, row, val)
    e3s = _sc_layer(e2s, col, row, val)
    light_out = _final(e0s, e1s, e2s, e3s)

    return light_out[:_N_USERS], light_out[_N_USERS:_N]
